# baseline (device time: 30453 ns/iter reference)
import jax
import jax.numpy as jnp
from jax import lax
from jax.experimental import pallas as pl
from jax.experimental.pallas import tpu as pltpu

_sem_signal = getattr(pl, "semaphore_signal", None) or pltpu.semaphore_signal
_sem_wait = getattr(pl, "semaphore_wait", None) or pltpu.semaphore_wait
_DeviceIdType = getattr(pl, "DeviceIdType", None) or pltpu.DeviceIdType
_CompilerParams = getattr(pltpu, "CompilerParams", None) or getattr(
    pltpu, "TPUCompilerParams"
)
_ANY = pl.MemorySpace.ANY

M = 1024
D = 1024
EPS = 1e-6
NB = 4
BM = M // NB
C = 4
CM = BM // C


def kernel(partial, resid, gamma):
    _HBM = pltpu.MemorySpace.HBM
    partial = pltpu.with_memory_space_constraint(partial, _HBM)
    resid = pltpu.with_memory_space_constraint(resid, _HBM)
    gamma = pltpu.with_memory_space_constraint(gamma, _HBM)

    def body(
        p_ref, r_ref, g_ref, out_ref,
        p_vm, r_vm, g_vm, stage, comm_x,
        in_sem, st_sem,
        sx_s, sx_r, yd_s, yd_r, zd_s, zd_r,
    ):
        my_x = lax.axis_index("x")
        my_y = lax.axis_index("y")
        my_z = lax.axis_index("z")
        xn = (1 - my_x, my_y, my_z)
        yn = (my_x, 1 - my_y, my_z)
        zn = (my_x, my_y, 1 - my_z)

        b = 2 * my_y + my_z
        bd = 3 - b

        def rows(blk, c):
            return pl.ds(blk * BM + c * CM, CM)

        def brows(blk):
            return pl.ds(blk * BM, BM)

        in_dma = [
            pltpu.make_async_copy(
                p_ref.at[0, brows(b), :], p_vm.at[pl.ds(0, BM), :],
                in_sem.at[0],
            ),
            pltpu.make_async_copy(
                p_ref.at[0, brows(bd), :], p_vm.at[pl.ds(BM, BM), :],
                in_sem.at[1],
            ),
            pltpu.make_async_copy(
                r_ref.at[brows(b), :], r_vm.at[pl.ds(0, BM), :],
                in_sem.at[2],
            ),
            pltpu.make_async_copy(
                r_ref.at[brows(bd), :], r_vm.at[pl.ds(BM, BM), :],
                in_sem.at[3],
            ),
            pltpu.make_async_copy(g_ref, g_vm, in_sem.at[4]),
        ]
        for d in in_dma:
            d.start()

        barrier_sem = pltpu.get_barrier_semaphore()
        for nbr in (xn, yn, zn):
            _sem_signal(
                barrier_sem, inc=1, device_id=nbr,
                device_id_type=_DeviceIdType.MESH,
            )
        _sem_wait(barrier_sem, 3)

        x_rdma = []
        for i in range(2 * C):
            blk, c = (b, i) if i < C else (bd, i - C)
            r = pltpu.make_async_remote_copy(
                src_ref=p_ref.at[0, rows(blk, c), :],
                dst_ref=comm_x.at[pl.ds(i * CM, CM), :],
                send_sem=sx_s.at[i],
                recv_sem=sx_r.at[i],
                device_id=xn,
                device_id_type=_DeviceIdType.MESH,
            )
            r.start()
            x_rdma.append(r)

        for d in in_dma:
            d.wait()

        st_dma = []

        def compute(blk, c, slot):
            y = p_vm[pl.ds(slot * CM, CM), :] + comm_x[pl.ds(slot * CM, CM), :] \
                + r_vm[pl.ds(slot * CM, CM), :]
            ms = jnp.mean(y * y, axis=-1, keepdims=True)
            stage[pl.ds(slot * CM, CM), :] = (
                y * lax.rsqrt(ms + EPS) * g_vm[...].reshape(1, D)
            )
            d = pltpu.make_async_copy(
                stage.at[pl.ds(slot * CM, CM), :],
                out_ref.at[rows(blk, c), :],
                st_sem.at[slot],
            )
            d.start()
            st_dma.append(d)

        yd, zd = [], []
        for c in range(C):
            x_rdma[c].wait_recv()
            compute(b, c, c)
            for sems_s, sems_r, nbr, acc in (
                (yd_s, yd_r, yn, yd),
                (zd_s, zd_r, zn, zd),
            ):
                r = pltpu.make_async_remote_copy(
                    src_ref=stage.at[pl.ds(c * CM, CM), :],
                    dst_ref=out_ref.at[rows(b, c), :],
                    send_sem=sems_s.at[c],
                    recv_sem=sems_r.at[c],
                    device_id=nbr,
                    device_id_type=_DeviceIdType.MESH,
                )
                r.start()
                acc.append(r)

        for c in range(C):
            x_rdma[C + c].wait_recv()
            compute(bd, c, C + c)

        for r in yd + zd:
            r.wait_recv()
        for d in st_dma:
            d.wait()
        for r in x_rdma + yd + zd:
            r.wait_send()

    return pl.pallas_call(
        body,
        out_shape=jax.ShapeDtypeStruct((M, D), jnp.float32),
        in_specs=[
            pl.BlockSpec(memory_space=pltpu.MemorySpace.HBM),
            pl.BlockSpec(memory_space=pltpu.MemorySpace.HBM),
            pl.BlockSpec(memory_space=pltpu.MemorySpace.HBM),
        ],
        out_specs=pl.BlockSpec(memory_space=pltpu.MemorySpace.HBM),
        scratch_shapes=[
            pltpu.VMEM((2 * BM, D), jnp.float32),
            pltpu.VMEM((2 * BM, D), jnp.float32),
            pltpu.VMEM((D,), jnp.float32),
            pltpu.VMEM((2 * BM, D), jnp.float32),
            pltpu.VMEM((2 * BM, D), jnp.float32),
            pltpu.SemaphoreType.DMA((5,)),
            pltpu.SemaphoreType.DMA((2 * C,)),
            pltpu.SemaphoreType.DMA((2 * C,)),
            pltpu.SemaphoreType.DMA((2 * C,)),
            pltpu.SemaphoreType.DMA((C,)),
            pltpu.SemaphoreType.DMA((C,)),
            pltpu.SemaphoreType.DMA((C,)),
            pltpu.SemaphoreType.DMA((C,)),
        ],
        compiler_params=_CompilerParams(collective_id=0),
    )(partial, resid, gamma)


# device time: 30208 ns/iter; 1.0081x vs baseline; 1.0081x over previous
import jax
import jax.numpy as jnp
from jax import lax
from jax.experimental import pallas as pl
from jax.experimental.pallas import tpu as pltpu

_sem_signal = getattr(pl, "semaphore_signal", None) or pltpu.semaphore_signal
_sem_wait = getattr(pl, "semaphore_wait", None) or pltpu.semaphore_wait
_DeviceIdType = getattr(pl, "DeviceIdType", None) or pltpu.DeviceIdType
_CompilerParams = getattr(pltpu, "CompilerParams", None) or getattr(
    pltpu, "TPUCompilerParams"
)
_HBM = pltpu.MemorySpace.HBM

M = 1024
D = 1024
EPS = 1e-6
NB = 4
BM = M // NB
C = 8
CM = BM // C
XB = 3
YF = (3, 4, 5)
ZF = (6, 7)


def kernel(partial, resid, gamma):
    partial = pltpu.with_memory_space_constraint(partial, _HBM)
    resid = pltpu.with_memory_space_constraint(resid, _HBM)
    gamma = pltpu.with_memory_space_constraint(gamma, _HBM)

    def body(
        p_ref, r_ref, g_ref, out_ref,
        p_vm, r_vm, g_vm, stage, comm_x,
        in_sem, st_sem,
        sx_s, sx_r, yd_s, yd_r, zd_s, zd_r, yf_s, yf_r, zf_s, zf_r,
    ):
        my_x = lax.axis_index("x")
        my_y = lax.axis_index("y")
        my_z = lax.axis_index("z")
        xn = (1 - my_x, my_y, my_z)
        yn = (my_x, 1 - my_y, my_z)
        zn = (my_x, my_y, 1 - my_z)

        b = 2 * my_y + my_z
        bd = 3 - b
        b_y = 2 * (1 - my_y) + my_z
        b_z = 2 * my_y + (1 - my_z)

        def rows(blk, c):
            return pl.ds(blk * BM + c * CM, CM)

        def brows(blk):
            return pl.ds(blk * BM, BM)

        def vrows(slot):
            return pl.ds(slot * CM, CM)

        in_dma = [
            pltpu.make_async_copy(
                p_ref.at[0, brows(b), :], p_vm.at[pl.ds(0, BM), :],
                in_sem.at[0],
            ),
            pltpu.make_async_copy(
                p_ref.at[0, brows(bd), :], p_vm.at[pl.ds(BM, BM), :],
                in_sem.at[1],
            ),
            pltpu.make_async_copy(
                r_ref.at[brows(b), :], r_vm.at[pl.ds(0, BM), :],
                in_sem.at[2],
            ),
            pltpu.make_async_copy(
                r_ref.at[brows(bd), :], r_vm.at[pl.ds(BM, BM), :],
                in_sem.at[3],
            ),
            pltpu.make_async_copy(g_ref, g_vm, in_sem.at[4]),
        ]
        for d in in_dma:
            d.start()

        barrier_sem = pltpu.get_barrier_semaphore()
        for nbr in (xn, yn, zn):
            _sem_signal(
                barrier_sem, inc=1, device_id=nbr,
                device_id_type=_DeviceIdType.MESH,
            )
        _sem_wait(barrier_sem, 3)

        x_rdma = []
        for i in range(C + XB):
            blk, c = (b, i) if i < C else (bd, i - C)
            r = pltpu.make_async_remote_copy(
                src_ref=p_ref.at[0, rows(blk, c), :],
                dst_ref=comm_x.at[vrows(i), :],
                send_sem=sx_s.at[i],
                recv_sem=sx_r.at[i],
                device_id=xn,
                device_id_type=_DeviceIdType.MESH,
            )
            r.start()
            x_rdma.append(r)

        for d in in_dma:
            d.wait()

        st_dma = []

        def compute(blk, c, slot):
            vs = pl.ds((0 if blk is b else BM) + c * CM, CM)
            y = p_vm[vs, :] + comm_x[vrows(slot), :] + r_vm[vs, :]
            ms = jnp.mean(y * y, axis=-1, keepdims=True)
            stage[vrows(slot), :] = (
                y * lax.rsqrt(ms + EPS) * g_vm[...].reshape(1, D)
            )
            d = pltpu.make_async_copy(
                stage.at[vrows(slot), :],
                out_ref.at[rows(blk, c), :],
                st_sem.at[slot],
            )
            d.start()
            st_dma.append(d)

        yd, zd = [], []
        for c in range(C):
            x_rdma[c].wait_recv()
            compute(b, c, c)
            for sems_s, sems_r, nbr, acc in (
                (yd_s, yd_r, yn, yd),
                (zd_s, zd_r, zn, zd),
            ):
                r = pltpu.make_async_remote_copy(
                    src_ref=stage.at[vrows(c), :],
                    dst_ref=out_ref.at[rows(b, c), :],
                    send_sem=sems_s.at[c],
                    recv_sem=sems_r.at[c],
                    device_id=nbr,
                    device_id_type=_DeviceIdType.MESH,
                )
                r.start()
                acc.append(r)

        for j in range(XB):
            x_rdma[C + j].wait_recv()
            compute(bd, j, C + j)

        yf, zf = [], []
        for k, j in enumerate(YF):
            zd[j].wait_recv()
            r = pltpu.make_async_remote_copy(
                src_ref=out_ref.at[rows(b_z, j), :],
                dst_ref=out_ref.at[rows(b_z, j), :],
                send_sem=yf_s.at[k],
                recv_sem=yf_r.at[k],
                device_id=yn,
                device_id_type=_DeviceIdType.MESH,
            )
            r.start()
            yf.append(r)
        for k, j in enumerate(ZF):
            yd[j].wait_recv()
            r = pltpu.make_async_remote_copy(
                src_ref=out_ref.at[rows(b_y, j), :],
                dst_ref=out_ref.at[rows(b_y, j), :],
                send_sem=zf_s.at[k],
                recv_sem=zf_r.at[k],
                device_id=zn,
                device_id_type=_DeviceIdType.MESH,
            )
            r.start()
            zf.append(r)

        for c in range(C):
            if c not in ZF:
                yd[c].wait_recv()
            if c not in YF:
                zd[c].wait_recv()
        for r in yf + zf:
            r.wait_recv()
        for d in st_dma:
            d.wait()
        for r in x_rdma + yd + zd + yf + zf:
            r.wait_send()

    return pl.pallas_call(
        body,
        out_shape=jax.ShapeDtypeStruct((M, D), jnp.float32),
        in_specs=[
            pl.BlockSpec(memory_space=_HBM),
            pl.BlockSpec(memory_space=_HBM),
            pl.BlockSpec(memory_space=_HBM),
        ],
        out_specs=pl.BlockSpec(memory_space=_HBM),
        scratch_shapes=[
            pltpu.VMEM((2 * BM, D), jnp.float32),
            pltpu.VMEM((2 * BM, D), jnp.float32),
            pltpu.VMEM((D,), jnp.float32),
            pltpu.VMEM(((C + XB) * CM, D), jnp.float32),
            pltpu.VMEM(((C + XB) * CM, D), jnp.float32),
            pltpu.SemaphoreType.DMA((5,)),
            pltpu.SemaphoreType.DMA((C + XB,)),
            pltpu.SemaphoreType.DMA((C + XB,)),
            pltpu.SemaphoreType.DMA((C + XB,)),
            pltpu.SemaphoreType.DMA((C,)),
            pltpu.SemaphoreType.DMA((C,)),
            pltpu.SemaphoreType.DMA((C,)),
            pltpu.SemaphoreType.DMA((C,)),
            pltpu.SemaphoreType.DMA((len(YF),)),
            pltpu.SemaphoreType.DMA((len(YF),)),
            pltpu.SemaphoreType.DMA((len(ZF),)),
            pltpu.SemaphoreType.DMA((len(ZF),)),
        ],
        compiler_params=_CompilerParams(collective_id=0),
    )(partial, resid, gamma)
